# baseline (speedup 1.0000x reference)
import functools

import jax
import jax.numpy as jnp
from jax import lax
from jax.experimental import pallas as pl
from jax.experimental.pallas import tpu as pltpu
from jax.experimental.pallas import tpu_sc as plsc

_B = 4
_C = 16
_H = 218
_NPOS = _H * _H
_NPOSP = 47616
_TILE = 512
_NT = _NPOSP // _TILE
_K = 8192
_NCB = 4096


def _cbsq_body(cb_ref, cs_ref):
    cbf = cb_ref[...]
    cs_ref[...] = jnp.sum(cbf * cbf, axis=1, keepdims=True)


def _cbsq(codebook):
    return pl.pallas_call(
        _cbsq_body,
        out_shape=jax.ShapeDtypeStruct((_K, 1), jnp.float32),
    )(codebook)


def _vq_body(z_ref, cs_ref, cbh_ref, idx_ref):
    zb = z_ref[0]
    zsq = jnp.sum(zb * zb, axis=0, keepdims=True)
    zh = zb.astype(jnp.bfloat16)

    cminv = []
    cidxv = []
    for k in range(_K // _NCB):
        s2 = lax.dot_general(cbh_ref[pl.ds(k * _NCB, _NCB), :], zh,
                             (((1,), (0,)), ((), ())),
                             preferred_element_type=jnp.float32)
        d = (zsq + s2) + cs_ref[pl.ds(k * _NCB, _NCB), :]
        cmin = jnp.min(d, axis=0, keepdims=True)
        rows = lax.broadcasted_iota(jnp.int32, (_NCB, _TILE), 0)
        cidx = jnp.min(jnp.where(d == cmin, rows, jnp.int32(2**30)), axis=0)
        cminv.append(cmin[0])
        cidxv.append(cidx + k * _NCB)

    best_d, best_i = cminv[0], cidxv[0]
    for k in range(1, _K // _NCB):
        take = cminv[k] < best_d
        best_d = jnp.where(take, cminv[k], best_d)
        best_i = jnp.where(take, cidxv[k], best_i)
    idx_ref[0, 0, 0] = best_i


def _vq_argmin(z_flat, codebook):
    return pl.pallas_call(
        _vq_body,
        grid=(_B, _NT),
        in_specs=[
            pl.BlockSpec((1, _C, _TILE), lambda b, t: (b, 0, t)),
            pl.BlockSpec((_K, 1), lambda b, t: (0, 0)),
            pl.BlockSpec((_K, _C), lambda b, t: (0, 0)),
        ],
        out_specs=pl.BlockSpec((1, 1, 1, _TILE), lambda b, t: (b, t, 0, 0)),
        out_shape=jax.ShapeDtypeStruct((_B, _NT, 1, _TILE), jnp.int32),
    )(z_flat, _cbsq(codebook),
      codebook.astype(jnp.bfloat16) * jnp.bfloat16(-2.0))


_NW = 32
_WPB = _NW // _B
_GCH = 49
_PW = _GCH * 128
_NPG = _WPB * _PW


def _gather_body(cb_hbm, idx_hbm, out_hbm, idx2v, rowsv, sem):
    wid = lax.axis_index("s") * 2 + lax.axis_index("c")
    base = wid * _PW
    pltpu.sync_copy(idx_hbm.at[pl.ds(wid * _GCH, _GCH), :], idx2v)

    cps = []
    for j in range(_GCH):
        cps.append(pltpu.async_copy(
            cb_hbm.at[idx2v.at[j]], rowsv.at[pl.ds(j * 128, 128), :], sem))
    for cp in cps:
        cp.wait()

    pltpu.sync_copy(rowsv, out_hbm.at[pl.ds(base, _PW), :])


def _sc_lookup(cb, idx_pad):
    mesh = plsc.VectorSubcoreMesh(core_axis_name="c", subcore_axis_name="s")
    return pl.kernel(
        _gather_body,
        out_type=jax.ShapeDtypeStruct((_B * _NPG, _C), jnp.float32),
        mesh=mesh,
        compiler_params=pltpu.CompilerParams(use_tc_tiling_on_sc=False),
        scratch_types=[
            pltpu.VMEM((_GCH, 128), jnp.int32),
            pltpu.VMEM((_PW, _C), jnp.float32),
            pltpu.SemaphoreType.DMA,
        ],
    )(cb, idx_pad.reshape(_NW * _GCH, 128))


def _conv(x, w, b):
    y = lax.conv_general_dilated(x, w, (1, 1), 'VALID',
                                 dimension_numbers=('NCHW', 'OIHW', 'NCHW'))
    return y + b[None, :, None, None]


_W224 = 224
_BUFW = 51456
_DQ = _NPG
_N1 = 220 * _W224
_N2 = 222 * _W224
_N3 = 224 * _W224
_WRB = 128 + 448

_SELU_L = 1.0507009873554804934193349852946
_SELU_A = 1.6732632423543772848170429916717


def _selu(x):
    return _SELU_L * jnp.where(x > 0, x, _SELU_A * (jnp.exp(x) - 1.0))


def _tr_body(zq_ref, out_ref):
    out_ref[0] = jnp.transpose(zq_ref[0])


def _transpose_rows(zq_rows):
    return pl.pallas_call(
        _tr_body,
        grid=(_B, 14),
        in_specs=[pl.BlockSpec((1, 3584, _C), lambda b, t: (b, t, 0))],
        out_specs=pl.BlockSpec((1, _C, 3584), lambda b, t: (b, 0, t)),
        out_shape=jax.ShapeDtypeStruct((_B, _C, _NPG), jnp.float32),
    )(zq_rows.reshape(_B, 14, 3584, _C).reshape(_B, _NPG, _C))


def _dec_body(zq_ref, d1_ref, d2_ref, d3_ref, b1_ref, b2_ref, b3_ref,
              out_ref, zqt_ref, g1_ref, g2_ref):
    @pl.when(pl.program_id(0) == 0)
    def _():
        zqt_ref[...] = jnp.zeros((_C, _BUFW), jnp.float32)
        g1_ref[...] = jnp.zeros((8, _BUFW), jnp.float32)
        g2_ref[...] = jnp.zeros((4, _BUFW), jnp.float32)

    zqt_ref[:, pl.ds(128, _DQ)] = zq_ref[0]

    def stage(src_ref, w_ref, b_ref, co, ci, n, base):
        acc = None
        for ky in range(3):
            for kx in range(3):
                wtap = w_ref[pl.ds((ky * 3 + kx) * co, co), :]
                slab = src_ref[:, pl.ds(base + ky * _W224 + kx - 1, n)]
                p = lax.dot_general(wtap, slab, (((1,), (0,)), ((), ())),
                                    preferred_element_type=jnp.float32,
                                    precision=lax.Precision.HIGHEST)
                acc = p if acc is None else acc + p
        return acc + b_ref[...]

    def colmask(n, lo, hi):
        col = lax.rem(lax.broadcasted_iota(jnp.int32, (1, n), 1),
                      jnp.int32(_W224))
        return ((col >= lo) & (col < hi)).astype(jnp.float32)

    g1 = _selu(stage(zqt_ref, d1_ref, b1_ref, 8, 16, _N1, 128))
    g1_ref[:, pl.ds(_WRB, _N1)] = g1 * colmask(_N1, 2, 222)
    g2 = _selu(stage(g1_ref, d2_ref, b2_ref, 4, 8, _N2, 128))
    g2_ref[:, pl.ds(_WRB, _N2)] = g2 * colmask(_N2, 1, 223)
    out_ref[0] = stage(g2_ref, d3_ref, b3_ref, 3, 4, _N3, 128)


def _decode(zq_rows, D1, d1, D2, d2, D3, d3):
    d1p = D1.transpose(2, 3, 1, 0).reshape(9 * 8, 16)
    d2p = D2.transpose(2, 3, 1, 0).reshape(9 * 4, 8)
    d3p = D3.transpose(2, 3, 1, 0).reshape(9 * 3, 4)
    recon = pl.pallas_call(
        _dec_body,
        grid=(_B,),
        in_specs=[
            pl.BlockSpec((1, _C, _NPG), lambda b: (b, 0, 0)),
            pl.BlockSpec((9 * 8, 16), lambda b: (0, 0)),
            pl.BlockSpec((9 * 4, 8), lambda b: (0, 0)),
            pl.BlockSpec((9 * 3, 4), lambda b: (0, 0)),
            pl.BlockSpec((8, 1), lambda b: (0, 0)),
            pl.BlockSpec((4, 1), lambda b: (0, 0)),
            pl.BlockSpec((3, 1), lambda b: (0, 0)),
        ],
        out_specs=pl.BlockSpec((1, 3, _N3), lambda b: (b, 0, 0)),
        out_shape=jax.ShapeDtypeStruct((_B, 3, _N3), jnp.float32),
        scratch_shapes=[
            pltpu.VMEM((_C, _BUFW), jnp.float32),
            pltpu.VMEM((8, _BUFW), jnp.float32),
            pltpu.VMEM((4, _BUFW), jnp.float32),
        ],
    )(_transpose_rows(zq_rows), d1p, d2p, d3p,
      d1.reshape(8, 1), d2.reshape(4, 1), d3.reshape(3, 1))
    return recon.reshape(_B, 3, _W224, _W224)


def kernel(x, W1, b1, W2, b2, W3, b3, D1, d1, D2, d2, D3, d3, codebook):
    h = jax.nn.selu(_conv(x, W1, b1))
    h = jax.nn.selu(_conv(h, W2, b2))
    z = _conv(h, W3, b3)

    z_flat = jnp.pad(z.reshape(_B, _C, _NPOS),
                     ((0, 0), (0, 0), (0, _NPOSP - _NPOS)))
    idx4 = _vq_argmin(z_flat, codebook)
    idx = idx4.reshape(_B, _NPOSP)[:, :_NPOS].reshape(_B, _H, _H)

    idx_dec = jnp.pad(idx, ((0, 0), (2, 2), (3, 3)), constant_values=_K)
    idx_dec = jnp.pad(idx_dec.reshape(_B, 222 * _W224), ((0, 0), (0, 448)),
                     constant_values=_K)
    cb_ext = jnp.concatenate([codebook, jnp.zeros((1, _C), jnp.float32)])
    zq_rows = _sc_lookup(cb_ext, idx_dec)

    recon = _decode(zq_rows, D1, d1, D2, d2, D3, d3)
    return recon, idx
